# baseline (speedup 1.0000x reference)
import functools

import jax
import jax.numpy as jnp
from jax.experimental import pallas as pl
from jax.experimental.pallas import tpu as pltpu


def _transpose8(t, masks):
    t = list(t)
    for d, mask in ((4, masks[0]), (2, masks[1]), (1, masks[2])):
        for i in range(8):
            if i & d:
                continue
            a, b = t[i], t[i ^ d]
            t[i] = jnp.where(mask, a, pltpu.roll(b, d, 0))
            t[i ^ d] = jnp.where(mask, pltpu.roll(a, 8 - d, 0), b)
    return t


def _route_row(xh, w, bias, eiota):
    logits = jnp.dot(w, xh, preferred_element_type=jnp.float32) + bias
    E = logits.shape[0]
    m1 = jnp.max(logits, axis=0, keepdims=True)
    i1 = jnp.min(jnp.where(logits == m1, eiota, E), axis=0, keepdims=True)
    masked = jnp.where(eiota == i1, -jnp.inf, logits)
    m2 = jnp.max(masked, axis=0, keepdims=True)
    i2 = jnp.min(jnp.where(masked == m2, eiota, E), axis=0, keepdims=True)
    ex = jnp.exp(logits - m1)
    recip = 1.0 / jnp.sum(ex, axis=0, keepdims=True)
    v1 = recip
    v2 = jnp.exp(m2 - m1) * recip
    return i1, i2, v1, v2


def _routing_body(x_ref, w_ref, b_ref, gates_ref, idx_ref, val_ref, xs_ref):
    w = w_ref[...]
    bias = b_ref[...]
    E = w.shape[0]
    C = x_ref.shape[1]
    Hb = x_ref.shape[2]
    Wd = x_ref.shape[3]
    eiota = jax.lax.broadcasted_iota(jnp.int32, (E, Wd), 0)
    siota = jax.lax.broadcasted_iota(jnp.int32, (Hb, Wd), 0)
    masks = [(siota & d) == 0 for d in (4, 2, 1)]

    for c8 in range(C // 8):
        tiles = [x_ref[0, c8 * 8 + j] for j in range(8)]
        rows = _transpose8(tiles, masks)
        for j in range(8):
            xs_ref[j, c8 * 8:(c8 + 1) * 8, :] = rows[j]

    i1s, i2s, v1s, v2s = [], [], [], []
    for j in range(Hb):
        i1, i2, v1, v2 = _route_row(xs_ref[j], w, bias, eiota)
        i1s.append(i1)
        i2s.append(i2)
        v1s.append(v1)
        v2s.append(v2)

    I1 = jnp.concatenate(i1s, axis=0)
    I2 = jnp.concatenate(i2s, axis=0)
    V1 = jnp.concatenate(v1s, axis=0)
    V2 = jnp.concatenate(v2s, axis=0)
    idx_ref[0, 0] = I1
    idx_ref[0, 1] = I2
    val_ref[0, 0] = V1
    val_ref[0, 1] = V2
    zero = jnp.zeros_like(V1)
    for e in range(E):
        gates_ref[0, e] = (jnp.where(I1 == e, V1, zero)
                           + jnp.where(I2 == e, V2, zero))


@functools.partial(jax.jit, static_argnames=())
def kernel(x, W, b):
    B, C, H, Wd = x.shape
    E = W.shape[0]
    b2 = b.reshape(E, 1)
    Hb = 8 if H % 8 == 0 else 1
    grid = (B, H // Hb)

    gates, idx, vals = pl.pallas_call(
        _routing_body,
        grid=grid,
        in_specs=[
            pl.BlockSpec((1, C, Hb, Wd), lambda bi, hi: (bi, 0, hi, 0)),
            pl.BlockSpec((E, C), lambda bi, hi: (0, 0)),
            pl.BlockSpec((E, 1), lambda bi, hi: (0, 0)),
        ],
        out_specs=[
            pl.BlockSpec((1, E, Hb, Wd), lambda bi, hi: (bi, 0, hi, 0)),
            pl.BlockSpec((1, 2, Hb, Wd), lambda bi, hi: (bi, 0, hi, 0)),
            pl.BlockSpec((1, 2, Hb, Wd), lambda bi, hi: (bi, 0, hi, 0)),
        ],
        out_shape=[
            jax.ShapeDtypeStruct((B, E, H, Wd), jnp.float32),
            jax.ShapeDtypeStruct((B, 2, H, Wd), jnp.int32),
            jax.ShapeDtypeStruct((B, 2, H, Wd), jnp.float32),
        ],
        scratch_shapes=[
            pltpu.VMEM((8, C, Wd), jnp.float32),
        ],
    )(x, W, b2)

    return gates, idx, vals


# speedup vs baseline: 1.0032x; 1.0032x over previous
import functools

import jax
import jax.numpy as jnp
from jax.experimental import pallas as pl
from jax.experimental.pallas import tpu as pltpu


def _transpose8(t, masks):
    t = list(t)
    for d, mask in ((4, masks[0]), (2, masks[1]), (1, masks[2])):
        for i in range(8):
            if i & d:
                continue
            a, b = t[i], t[i ^ d]
            t[i] = jnp.where(mask, a, pltpu.roll(b, d, 0))
            t[i ^ d] = jnp.where(mask, pltpu.roll(a, 8 - d, 0), b)
    return t


def _routing_body(x_ref, w_ref, b_ref, gates_ref, idx_ref, val_ref, xs_ref):
    w = w_ref[...]
    bias = b_ref[...]
    E = w.shape[0]
    C = x_ref.shape[1]
    Hb = x_ref.shape[2]
    Wd = x_ref.shape[3]
    Wp = ((Wd + 127) // 128) * 128
    siota = jax.lax.broadcasted_iota(jnp.int32, (Hb, Wd), 0)
    masks = [(siota & d) == 0 for d in (4, 2, 1)]

    for c8 in range(C // 8):
        tiles = [x_ref[0, c8 * 8 + j] for j in range(8)]
        rows = _transpose8(tiles, masks)
        for j in range(8):
            xs_ref[c8 * 8:(c8 + 1) * 8, j * Wp:j * Wp + Wd] = rows[j]

    logits = jnp.dot(w, xs_ref[...], preferred_element_type=jnp.float32)
    logits = logits + bias
    T = logits.shape[1]
    eiota = jax.lax.broadcasted_iota(jnp.int32, (E, T), 0)
    m1 = jnp.max(logits, axis=0, keepdims=True)
    i1 = jnp.min(jnp.where(logits == m1, eiota, E), axis=0, keepdims=True)
    masked = jnp.where(eiota == i1, -jnp.inf, logits)
    m2 = jnp.max(masked, axis=0, keepdims=True)
    i2 = jnp.min(jnp.where(masked == m2, eiota, E), axis=0, keepdims=True)
    ex = jnp.exp(logits - m1)
    recip = 1.0 / jnp.sum(ex, axis=0, keepdims=True)
    v1 = recip
    v2 = jnp.exp(m2 - m1) * recip

    def rows_of(v):
        return jnp.concatenate(
            [v[:, j * Wp:j * Wp + Wd] for j in range(Hb)], axis=0)

    I1 = rows_of(i1)
    I2 = rows_of(i2)
    V1 = rows_of(v1)
    V2 = rows_of(v2)
    idx_ref[0, 0] = I1
    idx_ref[0, 1] = I2
    val_ref[0, 0] = V1
    val_ref[0, 1] = V2
    zero = jnp.zeros_like(V1)
    for e in range(E):
        gates_ref[0, e] = (jnp.where(I1 == e, V1, zero)
                           + jnp.where(I2 == e, V2, zero))


@functools.partial(jax.jit, static_argnames=())
def kernel(x, W, b):
    B, C, H, Wd = x.shape
    E = W.shape[0]
    b2 = b.reshape(E, 1)
    Hb = 8 if H % 8 == 0 else 1
    Wp = ((Wd + 127) // 128) * 128
    grid = (B, H // Hb)

    gates, idx, vals = pl.pallas_call(
        _routing_body,
        grid=grid,
        in_specs=[
            pl.BlockSpec((1, C, Hb, Wd), lambda bi, hi: (bi, 0, hi, 0)),
            pl.BlockSpec((E, C), lambda bi, hi: (0, 0)),
            pl.BlockSpec((E, 1), lambda bi, hi: (0, 0)),
        ],
        out_specs=[
            pl.BlockSpec((1, E, Hb, Wd), lambda bi, hi: (bi, 0, hi, 0)),
            pl.BlockSpec((1, 2, Hb, Wd), lambda bi, hi: (bi, 0, hi, 0)),
            pl.BlockSpec((1, 2, Hb, Wd), lambda bi, hi: (bi, 0, hi, 0)),
        ],
        out_shape=[
            jax.ShapeDtypeStruct((B, E, H, Wd), jnp.float32),
            jax.ShapeDtypeStruct((B, 2, H, Wd), jnp.int32),
            jax.ShapeDtypeStruct((B, 2, H, Wd), jnp.float32),
        ],
        scratch_shapes=[
            pltpu.VMEM((C, Hb * Wp), jnp.float32),
        ],
    )(x, W, b2)

    return gates, idx, vals


# speedup vs baseline: 1.3384x; 1.3341x over previous
import functools

import jax
import jax.numpy as jnp
from jax.experimental import pallas as pl


def _to_rows(v, hb, wd):
    return jnp.concatenate([v[:, j * wd:(j + 1) * wd] for j in range(hb)],
                           axis=0)


def _routing_body(x_ref, w_ref, b_ref, gates_ref, idx_ref, val_ref):
    w = w_ref[...]
    bias = b_ref[...]
    E = w.shape[0]
    Hb = gates_ref.shape[2]
    Wd = gates_ref.shape[3]
    xb = x_ref[0]

    logits = jnp.dot(w, xb, preferred_element_type=jnp.float32) + bias
    T = logits.shape[1]
    eiota = jax.lax.broadcasted_iota(jnp.int32, (E, T), 0)
    m1 = jnp.max(logits, axis=0, keepdims=True)
    i1 = jnp.min(jnp.where(logits == m1, eiota, E), axis=0, keepdims=True)
    masked = jnp.where(eiota == i1, -jnp.inf, logits)
    m2 = jnp.max(masked, axis=0, keepdims=True)
    i2 = jnp.min(jnp.where(masked == m2, eiota, E), axis=0, keepdims=True)
    ex = jnp.exp(logits - m1)
    recip = 1.0 / jnp.sum(ex, axis=0, keepdims=True)
    v1 = recip
    v2 = jnp.exp(m2 - m1) * recip

    I1 = _to_rows(i1, Hb, Wd)
    I2 = _to_rows(i2, Hb, Wd)
    V1 = _to_rows(v1, Hb, Wd)
    V2 = _to_rows(v2, Hb, Wd)
    idx_ref[0, 0] = I1
    idx_ref[0, 1] = I2
    val_ref[0, 0] = V1
    val_ref[0, 1] = V2
    zero = jnp.zeros_like(V1)
    for e in range(E):
        gates_ref[0, e] = (jnp.where(I1 == e, V1, zero)
                           + jnp.where(I2 == e, V2, zero))


@functools.partial(jax.jit, static_argnames=())
def kernel(x, W, b):
    B, C, H, Wd = x.shape
    E = W.shape[0]
    S = H * Wd
    xr = x.reshape(B, C, S)
    b2 = b.reshape(E, 1)
    Hb = 16 if H % 16 == 0 else (8 if H % 8 == 0 else 1)
    T = Hb * Wd
    grid = (B, H // Hb)

    gates, idx, vals = pl.pallas_call(
        _routing_body,
        grid=grid,
        in_specs=[
            pl.BlockSpec((1, C, T), lambda bi, hi: (bi, 0, hi)),
            pl.BlockSpec((E, C), lambda bi, hi: (0, 0)),
            pl.BlockSpec((E, 1), lambda bi, hi: (0, 0)),
        ],
        out_specs=[
            pl.BlockSpec((1, E, Hb, Wd), lambda bi, hi: (bi, 0, hi, 0)),
            pl.BlockSpec((1, 2, Hb, Wd), lambda bi, hi: (bi, 0, hi, 0)),
            pl.BlockSpec((1, 2, Hb, Wd), lambda bi, hi: (bi, 0, hi, 0)),
        ],
        out_shape=[
            jax.ShapeDtypeStruct((B, E, H, Wd), jnp.float32),
            jax.ShapeDtypeStruct((B, 2, H, Wd), jnp.int32),
            jax.ShapeDtypeStruct((B, 2, H, Wd), jnp.float32),
        ],
    )(xr, W, b2)

    return gates, idx, vals


# speedup vs baseline: 1.3837x; 1.0339x over previous
import functools

import jax
import jax.numpy as jnp
from jax.experimental import pallas as pl


def _to_rows(v, hb, wd):
    return jnp.concatenate([v[:, j * wd:(j + 1) * wd] for j in range(hb)],
                           axis=0)


def _routing_body(x_ref, w_ref, b_ref, gates_ref, idx_ref, val_ref):
    w = w_ref[...]
    bias = b_ref[...]
    E = w.shape[0]
    Hb = gates_ref.shape[2]
    Wd = gates_ref.shape[3]
    xb = x_ref[0]

    logits = jnp.dot(w, xb, preferred_element_type=jnp.float32) + bias
    T = logits.shape[1]
    eiota = jax.lax.broadcasted_iota(jnp.int32, (E, T), 0)
    m1 = jnp.max(logits, axis=0, keepdims=True)
    i1 = jnp.min(jnp.where(logits == m1, eiota, E), axis=0, keepdims=True)
    masked = jnp.where(eiota == i1, -jnp.inf, logits)
    m2 = jnp.max(masked, axis=0, keepdims=True)
    i2 = jnp.min(jnp.where(masked == m2, eiota, E), axis=0, keepdims=True)
    ex = jnp.exp(logits - m1)
    recip = 1.0 / jnp.sum(ex, axis=0, keepdims=True)
    v1 = recip
    v2 = jnp.exp(m2 - m1) * recip

    I1 = _to_rows(i1, Hb, Wd)
    I2 = _to_rows(i2, Hb, Wd)
    V1 = _to_rows(v1, Hb, Wd)
    V2 = _to_rows(v2, Hb, Wd)
    idx_ref[0, 0] = I1
    idx_ref[0, 1] = I2
    val_ref[0, 0] = V1
    val_ref[0, 1] = V2
    zero = jnp.zeros_like(V1)
    for e in range(E):
        gates_ref[0, e] = (jnp.where(I1 == e, V1, zero)
                           + jnp.where(I2 == e, V2, zero))


@functools.partial(jax.jit, static_argnames=())
def kernel(x, W, b):
    B, C, H, Wd = x.shape
    E = W.shape[0]
    S = H * Wd
    xr = x.reshape(B, C, S)
    b2 = b.reshape(E, 1)
    Hb = 32 if H % 32 == 0 else (8 if H % 8 == 0 else 1)
    T = Hb * Wd
    grid = (B, H // Hb)

    gates, idx, vals = pl.pallas_call(
        _routing_body,
        grid=grid,
        in_specs=[
            pl.BlockSpec((1, C, T), lambda bi, hi: (bi, 0, hi)),
            pl.BlockSpec((E, C), lambda bi, hi: (0, 0)),
            pl.BlockSpec((E, 1), lambda bi, hi: (0, 0)),
        ],
        out_specs=[
            pl.BlockSpec((1, E, Hb, Wd), lambda bi, hi: (bi, 0, hi, 0)),
            pl.BlockSpec((1, 2, Hb, Wd), lambda bi, hi: (bi, 0, hi, 0)),
            pl.BlockSpec((1, 2, Hb, Wd), lambda bi, hi: (bi, 0, hi, 0)),
        ],
        out_shape=[
            jax.ShapeDtypeStruct((B, E, H, Wd), jnp.float32),
            jax.ShapeDtypeStruct((B, 2, H, Wd), jnp.int32),
            jax.ShapeDtypeStruct((B, 2, H, Wd), jnp.float32),
        ],
    )(xr, W, b2)

    return gates, idx, vals


# speedup vs baseline: 1.3994x; 1.0113x over previous
import functools

import jax
import jax.numpy as jnp
from jax.experimental import pallas as pl


def _to_rows(v, hb, wd):
    return jnp.concatenate([v[:, j * wd:(j + 1) * wd] for j in range(hb)],
                           axis=0)


def _routing_body(x_ref, w_ref, b_ref, gates_ref, idx_ref, val_ref):
    w = w_ref[...]
    bias = b_ref[...]
    E = w.shape[0]
    Hb = gates_ref.shape[2]
    Wd = gates_ref.shape[3]
    xb = x_ref[0]

    logits = jnp.dot(w, xb, preferred_element_type=jnp.float32) + bias
    T = logits.shape[1]
    eiota = jax.lax.broadcasted_iota(jnp.int32, (E, T), 0)
    m1 = jnp.max(logits, axis=0, keepdims=True)
    i1 = jnp.min(jnp.where(logits == m1, eiota, E), axis=0, keepdims=True)
    masked = jnp.where(eiota == i1, -jnp.inf, logits)
    m2 = jnp.max(masked, axis=0, keepdims=True)
    i2 = jnp.min(jnp.where(masked == m2, eiota, E), axis=0, keepdims=True)
    ex = jnp.exp(logits - m1)
    recip = 1.0 / jnp.sum(ex, axis=0, keepdims=True)
    v1 = recip
    v2 = jnp.exp(m2 - m1) * recip

    I1 = _to_rows(i1, Hb, Wd)
    I2 = _to_rows(i2, Hb, Wd)
    V1 = _to_rows(v1, Hb, Wd)
    V2 = _to_rows(v2, Hb, Wd)
    idx_ref[0, 0] = I1
    idx_ref[0, 1] = I2
    val_ref[0, 0] = V1
    val_ref[0, 1] = V2
    zero = jnp.zeros_like(V1)
    for e in range(E):
        gates_ref[0, e] = (jnp.where(I1 == e, V1, zero)
                           + jnp.where(I2 == e, V2, zero))


@functools.partial(jax.jit, static_argnames=())
def kernel(x, W, b):
    B, C, H, Wd = x.shape
    E = W.shape[0]
    S = H * Wd
    xr = x.reshape(B, C, S)
    b2 = b.reshape(E, 1)
    Hb = 56 if H % 56 == 0 else (8 if H % 8 == 0 else 1)
    T = Hb * Wd
    grid = (B, H // Hb)

    gates, idx, vals = pl.pallas_call(
        _routing_body,
        grid=grid,
        in_specs=[
            pl.BlockSpec((1, C, T), lambda bi, hi: (bi, 0, hi)),
            pl.BlockSpec((E, C), lambda bi, hi: (0, 0)),
            pl.BlockSpec((E, 1), lambda bi, hi: (0, 0)),
        ],
        out_specs=[
            pl.BlockSpec((1, E, Hb, Wd), lambda bi, hi: (bi, 0, hi, 0)),
            pl.BlockSpec((1, 2, Hb, Wd), lambda bi, hi: (bi, 0, hi, 0)),
            pl.BlockSpec((1, 2, Hb, Wd), lambda bi, hi: (bi, 0, hi, 0)),
        ],
        out_shape=[
            jax.ShapeDtypeStruct((B, E, H, Wd), jnp.float32),
            jax.ShapeDtypeStruct((B, 2, H, Wd), jnp.int32),
            jax.ShapeDtypeStruct((B, 2, H, Wd), jnp.float32),
        ],
    )(xr, W, b2)

    return gates, idx, vals
